# baseline (speedup 1.0000x reference)
import functools

import jax
import jax.numpy as jnp
from jax import lax
from jax.experimental import pallas as pl
from jax.experimental.pallas import tpu as pltpu
from jax.experimental.pallas import tpu_sc as plsc

N = 10000
NP = 10240
D = 128
E = 320000
G = 16
NLAYER = 3
NC = 2
NS = 16
NW = NC * NS
KB = 80
EB = E // NW
NBE = EB // KB
ROWS_T = NP // NS
NODE_T = NP // NW
NBN = NODE_T // KB
DUMMY = N
BM = 1024
NEG = -1e30


def _zero_rows(buf, nrow, width):
    def zrow(r, carry):
        for jj in range(width // 16):
            buf[r, pl.ds(jj * 16, 16)] = jnp.zeros((16,), jnp.float32)
        return carry
    lax.fori_loop(0, nrow, zrow, 0)


def _spmm_body(x_hbm, srcs_hbm, dsts_hbm, out_hbm, src_v, dst_v, rows_v, acc,
               sem):
    c = lax.axis_index("c")
    s = lax.axis_index("s")
    wid = s * NC + c

    _zero_rows(rows_v, KB, D)
    def zcp(t, carry):
        pltpu.sync_copy(rows_v, acc.at[pl.ds(s * ROWS_T + t * KB, KB)])
        return carry
    lax.fori_loop(0, ROWS_T // KB, zcp, 0)
    plsc.subcore_barrier()

    pltpu.sync_copy(srcs_hbm.at[wid], src_v)
    pltpu.sync_copy(dsts_hbm.at[wid], dst_v)

    def step(j, carry):
        pltpu.async_copy(x_hbm.at[src_v.at[j]], rows_v, sem).wait()
        pltpu.sync_copy(rows_v, acc.at[dst_v.at[j]], add=True)
        return carry
    lax.fori_loop(0, NBE, step, 0)
    plsc.subcore_barrier()
    pltpu.sync_copy(acc.at[pl.ds(s * ROWS_T, ROWS_T)],
                    out_hbm.at[c, pl.ds(s * ROWS_T, ROWS_T)])


@functools.cache
def _get_spmm():
    return pl.kernel(
        _spmm_body,
        out_type=jax.ShapeDtypeStruct((NC, NP, D), jnp.float32),
        mesh=plsc.VectorSubcoreMesh(core_axis_name="c", subcore_axis_name="s"),
        scratch_types=[
            pltpu.VMEM((NBE, KB), jnp.int32),
            pltpu.VMEM((NBE, KB), jnp.int32),
            pltpu.VMEM((KB, D), jnp.float32),
            pltpu.VMEM_SHARED((NP, D), jnp.float32),
            pltpu.SemaphoreType.DMA,
        ],
    )


def _spmm(x, srcs, dsts):
    return _get_spmm()(x, srcs, dsts)


def _gath_body(ct_hbm, idx_hbm, out_hbm, idx_v, r0, r1, r2, sem0, sem1, sem2):
    c = lax.axis_index("c")
    s = lax.axis_index("s")
    wid = s * NC + c

    pltpu.sync_copy(idx_hbm.at[wid], idx_v)
    for j in range(NBN):
        cp0 = pltpu.async_copy(ct_hbm.at[idx_v.at[0, j]], r0, sem0)
        cp1 = pltpu.async_copy(ct_hbm.at[idx_v.at[1, j]], r1, sem1)
        cp2 = pltpu.async_copy(ct_hbm.at[idx_v.at[2, j]], r2, sem2)
        cp0.wait()
        cp1.wait()
        cp2.wait()
        def addrow(r, carry):
            for jj in range(D // 16):
                sl = pl.ds(jj * 16, 16)
                r0[r, sl] = r0[r, sl] + r1[r, sl] + r2[r, sl]
            return carry
        lax.fori_loop(0, KB, addrow, 0)
        pltpu.sync_copy(r0, out_hbm.at[pl.ds(wid * NODE_T + j * KB, KB)])


@functools.cache
def _get_gath():
    return pl.kernel(
        _gath_body,
        out_type=jax.ShapeDtypeStruct((NP, D), jnp.float32),
        mesh=plsc.VectorSubcoreMesh(core_axis_name="c", subcore_axis_name="s"),
        scratch_types=[
            pltpu.VMEM((3, NBN, KB), jnp.int32),
            pltpu.VMEM((KB, D), jnp.float32),
            pltpu.VMEM((KB, D), jnp.float32),
            pltpu.VMEM((KB, D), jnp.float32),
            pltpu.SemaphoreType.DMA,
            pltpu.SemaphoreType.DMA,
            pltpu.SemaphoreType.DMA,
        ],
    )


def _gath(ct, idx3):
    return _get_gath()(ct, idx3)


_GRID = NP // BM


def _tables_body(ide, cate, idxe, w1, w2, w3, ct):
    ct[0:256, :] = jnp.dot(jnp.maximum(ide[...], 0.0), w1[...],
                           preferred_element_type=jnp.float32)
    ct[256:296, :] = jnp.dot(jnp.maximum(cate[...], 0.0), w2[...],
                             preferred_element_type=jnp.float32)
    ct[296:3360, :] = jnp.dot(jnp.maximum(idxe[...], 0.0), w3[...],
                              preferred_element_type=jnp.float32)


def _node_body(pf, pw, pb, gath, nw, nb, hist, node, dinv):
    pos = jnp.maximum(
        jnp.dot(pf[...], pw[...], preferred_element_type=jnp.float32)
        + pb[...], 0.0)
    node[...] = jnp.maximum(
        jnp.dot(pos, nw[...], preferred_element_type=jnp.float32)
        + gath[...] + nb[...], 0.0)
    deg = 1.0 + hist[0, :, 0:1] + hist[1, :, 0:1]
    dinv[...] = jnp.broadcast_to(lax.rsqrt(deg), (BM, 8))


def _mm_scale_body(x, w, dinv, o):
    h = jnp.dot(x[...], w[...], preferred_element_type=jnp.float32)
    o[...] = h * dinv[:, 0:1]


def _gcn_post_body(p, hs, dinv, b, o):
    q = p[0] + p[1] + hs[...]
    o[...] = jnp.maximum(q * dinv[:, 0:1] + b[...], 0.0)


def _gin_body(gcn, p, w1, b1, w2, b2, o):
    hg = gcn[...] + p[0] + p[1]
    y = jnp.maximum(
        jnp.dot(hg, w1[...], preferred_element_type=jnp.float32) + b1[...],
        0.0)
    o[...] = jnp.maximum(
        jnp.dot(y, w2[...], preferred_element_type=jnp.float32) + b2[...],
        0.0)


def _pool_body(x, bb, o):
    @pl.when(pl.program_id(0) == 0)
    def _init():
        o[...] = jnp.full((G, D), -jnp.inf, jnp.float32)
    xv = x[...]
    b = bb[:, 0:1]
    rows = []
    for g in range(G):
        v = jnp.max(jnp.where(b == g, xv, -jnp.inf), axis=0, keepdims=True)
        rows.append(v)
    o[...] = jnp.maximum(o[...], jnp.concatenate(rows, axis=0))


def _head_body(ge, agW, agb, pdW, pdb, pfW, pfb, idW, idb, ifW, ifb,
               cdW, cdb, cfW, cfb, pos_o, id_o, cat_o):
    def mm(a, b):
        return jnp.dot(a, b[...], preferred_element_type=jnp.float32)

    latent = mm(ge[...], agW) + agb[...]
    t = jnp.maximum(mm(latent, pdW) + pdb[...], 0.0)
    z = mm(t, pfW) + pfb[...]
    pos_o[...] = 1.0 / (1.0 + jnp.exp(-z))
    t = jnp.maximum(mm(latent, idW) + idb[...], 0.0)
    z = mm(t, ifW) + ifb[...]
    e = jnp.exp(z - jnp.max(z, axis=-1, keepdims=True))
    id_o[...] = e / jnp.sum(e, axis=-1, keepdims=True)
    t = jnp.maximum(mm(latent, cdW) + cdb[...], 0.0)
    z = mm(t, cfW) + cfb[...]
    e = jnp.exp(z - jnp.max(z, axis=-1, keepdims=True))
    cat_o[...] = e / jnp.sum(e, axis=-1, keepdims=True)


def _row_spec(shape):
    return pl.BlockSpec(shape, lambda i: (i,) + (0,) * (len(shape) - 1))


def _fix_spec(shape):
    return pl.BlockSpec(shape, lambda i: (0,) * len(shape))


def _tables(ide, cate, idxe, w1, w2, w3):
    return pl.pallas_call(
        _tables_body,
        out_shape=jax.ShapeDtypeStruct((3360, D), jnp.float32),
    )(ide, cate, idxe, w1, w2, w3)


def _node(pf, pw, pb, gath, nw, nb, hist):
    return pl.pallas_call(
        _node_body,
        grid=(_GRID,),
        in_specs=[
            _row_spec((BM, 8)), _fix_spec((8, D)), _fix_spec((1, D)),
            _row_spec((BM, D)), _fix_spec((D, D)), _fix_spec((1, D)),
            pl.BlockSpec((NC, BM, D), lambda i: (0, i, 0)),
        ],
        out_specs=[_row_spec((BM, D)), _row_spec((BM, 8))],
        out_shape=[jax.ShapeDtypeStruct((NP, D), jnp.float32),
                   jax.ShapeDtypeStruct((NP, 8), jnp.float32)],
    )(pf, pw, pb, gath, nw, nb, hist)


def _mm_scale(x, w, dinv):
    return pl.pallas_call(
        _mm_scale_body,
        grid=(_GRID,),
        in_specs=[_row_spec((BM, D)), _fix_spec((D, D)), _row_spec((BM, 8))],
        out_specs=_row_spec((BM, D)),
        out_shape=jax.ShapeDtypeStruct((NP, D), jnp.float32),
    )(x, w, dinv)


def _gcn_post(p, hs, dinv, b):
    return pl.pallas_call(
        _gcn_post_body,
        grid=(_GRID,),
        in_specs=[
            pl.BlockSpec((NC, BM, D), lambda i: (0, i, 0)),
            _row_spec((BM, D)), _row_spec((BM, 8)), _fix_spec((1, D)),
        ],
        out_specs=_row_spec((BM, D)),
        out_shape=jax.ShapeDtypeStruct((NP, D), jnp.float32),
    )(p, hs, dinv, b)


def _gin(gcn, p, w1, b1, w2, b2):
    return pl.pallas_call(
        _gin_body,
        grid=(_GRID,),
        in_specs=[
            _row_spec((BM, D)),
            pl.BlockSpec((NC, BM, D), lambda i: (0, i, 0)),
            _fix_spec((D, D)), _fix_spec((1, D)),
            _fix_spec((D, D)), _fix_spec((1, D)),
        ],
        out_specs=_row_spec((BM, D)),
        out_shape=jax.ShapeDtypeStruct((NP, D), jnp.float32),
    )(gcn, p, w1, b1, w2, b2)


def _pool(x, bb):
    return pl.pallas_call(
        _pool_body,
        grid=(_GRID,),
        in_specs=[_row_spec((BM, D)), _row_spec((BM, 8))],
        out_specs=_fix_spec((G, D)),
        out_shape=jax.ShapeDtypeStruct((G, D), jnp.float32),
    )(x, bb)


def _head(ge, agW, agb, pdW, pdb, pfW, pfb, idW, idb, ifW, ifb,
          cdW, cdb, cfW, cfb):
    return pl.pallas_call(
        _head_body,
        out_shape=[jax.ShapeDtypeStruct((G, 8), jnp.float32),
                   jax.ShapeDtypeStruct((G, 256), jnp.float32),
                   jax.ShapeDtypeStruct((G, D), jnp.float32)],
    )(ge, agW, agb, pdW, pdb, pfW, pfb, idW, idb, ifW, ifb,
      cdW, cdb, cfW, cfb)


def kernel(position_feature, id_feature, category_feature, idx_feature,
           edge_index, edge_attr, batch, pW, pb, id_emb, cat_emb, idx_emb,
           neW, neb, tW, tb, gW1, gb1, gW2, gb2, agW, agb, pdW, pdb, pfW,
           pfb, idW, idb, ifW, ifb, cdW, cdb, cfW, cfb):
    i32 = jnp.int32
    src = edge_index[0].astype(i32)
    dst = edge_index[1].astype(i32)
    act = edge_attr.astype(i32) == 1
    src_t = jnp.where(act, src, 0).reshape(NW, NBE, KB)
    dst_t = jnp.where(act, dst, DUMMY).reshape(NW, NBE, KB)
    src_s = jnp.where(act, 0, src).reshape(NW, NBE, KB)
    dst_s = jnp.where(act, DUMMY, dst).reshape(NW, NBE, KB)

    idx3 = jnp.stack([id_feature.astype(i32),
                      category_feature.astype(i32) + 256,
                      idx_feature.astype(i32) + 296])
    idx3 = jnp.pad(idx3, ((0, 0), (0, NP - N)))
    idx3 = idx3.reshape(3, NW, NBN, KB).transpose(1, 0, 2, 3)

    bb8 = jnp.broadcast_to(
        jnp.pad(batch.astype(i32), (0, NP - N), constant_values=G)[:, None],
        (NP, 8))
    pf8 = jnp.pad(position_feature, ((0, NP - N), (0, 5)))
    pW8 = jnp.pad(pW, ((0, 5), (0, 0)))
    cat_emb_p = jnp.pad(cat_emb, ((0, 6), (0, 0)))
    idx_emb_p = jnp.pad(idx_emb, ((0, 4), (0, 0)))
    pfW8 = jnp.pad(pfW, ((0, 0), (0, 5)))
    pfb8 = jnp.pad(pfb, (0, 5))
    cfW128 = jnp.pad(cfW, ((0, 0), (0, D - 34)))
    cfb128 = jnp.pad(cfb, (0, D - 34), constant_values=NEG)

    histp = _spmm(jnp.ones((NP, D), jnp.float32), src_t, dst_t)
    ct = _tables(id_emb, cat_emb_p, idx_emb_p,
                 neW[128:256], neW[256:384], neW[384:512])
    gath = _gath(ct, idx3)
    node, dinv8 = _node(pf8, pW8, pb[None], gath, neW[0:128], neb[None],
                        histp)

    pools = [_pool(node, bb8)]
    x = node
    for i in range(NLAYER):
        hs = _mm_scale(x, tW[i], dinv8)
        p = _spmm(hs, src_t, dst_t)
        gcn = _gcn_post(p, hs, dinv8, tb[i][None])
        q = _spmm(gcn, src_s, dst_s)
        nt = _gin(gcn, q, gW1[i], gb1[i][None], gW2[i], gb2[i][None])
        pools.append(_pool(nt, bb8))
        x = nt

    ge = jnp.concatenate(pools, axis=1)
    pos8, id_out, cat128 = _head(
        ge, agW, agb[None], pdW, pdb[None], pfW8, pfb8[None],
        idW, idb[None], ifW, ifb[None], cdW, cdb[None], cfW128, cfb128[None])
    return pos8[:, :3], id_out, cat128[:, :34]


# speedup vs baseline: 12.1906x; 12.1906x over previous
import functools

import jax
import jax.numpy as jnp
from jax import lax
from jax.experimental import pallas as pl
from jax.experimental.pallas import tpu as pltpu
from jax.experimental.pallas import tpu_sc as plsc

N = 10000
NP = 10240
D = 128
E = 320000
G = 16
NLAYER = 3
NC = 2
NS = 16
NW = NC * NS
KB = 128
NBE = 80
CHB = 8
NCH = NBE // CHB
EBP = NBE * KB
EP = NW * EBP
NBUF = 2
ROWS_T = NP // NS
NODE_T = NP // NW
KGN = 80
NBN = NODE_T // KGN
DUMMY = N
BM = 1024
NEG = -1e30


def _zero_rows(buf, nrow, width):
    def zrow(r, carry):
        for jj in range(width // 16):
            buf[r, pl.ds(jj * 16, 16)] = jnp.zeros((16,), jnp.float32)
        return carry
    lax.fori_loop(0, nrow, zrow, 0)


def _spmm_body(x_hbm, srcs_hbm, dsts_hbm, out_hbm,
               sa, sb, da, db, r0, r1, acc,
               ia, ib, g0, g1, s0, s1):
    c = lax.axis_index("c")
    s = lax.axis_index("s")
    wid = s * NC + c
    sbank = (sa, sb)
    dbank = (da, db)
    isems = (ia, ib)
    bufs = (r0, r1)
    gsems = (g0, g1)
    ssems = (s0, s1)

    _zero_rows(r0, KB, D)
    def zcp(t, carry):
        pltpu.sync_copy(r0, acc.at[pl.ds(s * ROWS_T + t * KB, KB)])
        return carry
    lax.fori_loop(0, ROWS_T // KB, zcp, 0)
    plsc.subcore_barrier()

    def fire_idx(k, bank):
        pltpu.async_copy(srcs_hbm.at[wid, pl.ds(k * CHB, CHB)],
                         sbank[bank], isems[bank])
        pltpu.async_copy(dsts_hbm.at[wid, pl.ds(k * CHB, CHB)],
                         dbank[bank], isems[bank])

    def wait_idx(bank):
        pltpu.make_async_copy(srcs_hbm.at[wid, pl.ds(0, CHB)],
                              sbank[bank], isems[bank]).wait()
        pltpu.make_async_copy(dsts_hbm.at[wid, pl.ds(0, CHB)],
                              dbank[bank], isems[bank]).wait()

    fire_idx(0, 0)

    def chunk_pair(cp, carry):
        for half in range(2):
            k = cp * 2 + half
            sbk = sbank[half]
            dbk = dbank[half]
            wait_idx(half)
            @pl.when(k + 1 < NCH)
            def _next_idx():
                fire_idx(k + 1, 1 - half)
            pltpu.async_copy(x_hbm.at[sbk.at[0]], bufs[0], gsems[0])
            pltpu.async_copy(x_hbm.at[sbk.at[1]], bufs[1], gsems[1])

            def ring(jl, rcarry):
                for b in range(NBUF):
                    jloc = jl * NBUF + b
                    pltpu.make_async_copy(x_hbm.at[sbk.at[jloc]], bufs[b],
                                          gsems[b]).wait()
                    pltpu.async_copy(bufs[b], acc.at[dbk.at[jloc]],
                                     ssems[b], add=True)
                for b in range(NBUF):
                    jloc = jl * NBUF + b
                    pltpu.make_async_copy(bufs[b], acc.at[dbk.at[jloc]],
                                          ssems[b]).wait()
                    @pl.when(jloc + NBUF < CHB)
                    def _prefetch():
                        pltpu.async_copy(x_hbm.at[sbk.at[jloc + NBUF]],
                                         bufs[b], gsems[b])
                return rcarry
            lax.fori_loop(0, CHB // NBUF, ring, 0)
        return carry
    lax.fori_loop(0, NCH // 2, chunk_pair, 0)
    plsc.subcore_barrier()
    pltpu.sync_copy(acc.at[pl.ds(s * ROWS_T, ROWS_T)],
                    out_hbm.at[c, pl.ds(s * ROWS_T, ROWS_T)])


@functools.cache
def _get_spmm():
    return pl.kernel(
        _spmm_body,
        out_type=jax.ShapeDtypeStruct((NC, NP, D), jnp.float32),
        mesh=plsc.VectorSubcoreMesh(core_axis_name="c", subcore_axis_name="s"),
        scratch_types=[
            pltpu.VMEM((CHB, KB), jnp.int32),
            pltpu.VMEM((CHB, KB), jnp.int32),
            pltpu.VMEM((CHB, KB), jnp.int32),
            pltpu.VMEM((CHB, KB), jnp.int32),
            pltpu.VMEM((KB, D), jnp.float32),
            pltpu.VMEM((KB, D), jnp.float32),
            pltpu.VMEM_SHARED((NP, D), jnp.float32),
            pltpu.SemaphoreType.DMA,
            pltpu.SemaphoreType.DMA,
            pltpu.SemaphoreType.DMA,
            pltpu.SemaphoreType.DMA,
            pltpu.SemaphoreType.DMA,
            pltpu.SemaphoreType.DMA,
        ],
    )


def _spmm(x, srcs, dsts):
    return _get_spmm()(x, srcs, dsts)


def _gath_body(ct_hbm, idx_hbm, out_hbm, idx_v, r0, r1, r2, sem0, sem1, sem2):
    c = lax.axis_index("c")
    s = lax.axis_index("s")
    wid = s * NC + c

    pltpu.sync_copy(idx_hbm.at[wid], idx_v)
    for j in range(NBN):
        cp0 = pltpu.async_copy(ct_hbm.at[idx_v.at[0, j]], r0, sem0)
        cp1 = pltpu.async_copy(ct_hbm.at[idx_v.at[1, j]], r1, sem1)
        cp2 = pltpu.async_copy(ct_hbm.at[idx_v.at[2, j]], r2, sem2)
        cp0.wait()
        cp1.wait()
        cp2.wait()
        def addrow(r, carry):
            for jj in range(D // 16):
                sl = pl.ds(jj * 16, 16)
                r0[r, sl] = r0[r, sl] + r1[r, sl] + r2[r, sl]
            return carry
        lax.fori_loop(0, KGN, addrow, 0)
        pltpu.sync_copy(r0, out_hbm.at[pl.ds(wid * NODE_T + j * KGN, KGN)])


@functools.cache
def _get_gath():
    return pl.kernel(
        _gath_body,
        out_type=jax.ShapeDtypeStruct((NP, D), jnp.float32),
        mesh=plsc.VectorSubcoreMesh(core_axis_name="c", subcore_axis_name="s"),
        scratch_types=[
            pltpu.VMEM((3, NBN, KGN), jnp.int32),
            pltpu.VMEM((KGN, D), jnp.float32),
            pltpu.VMEM((KGN, D), jnp.float32),
            pltpu.VMEM((KGN, D), jnp.float32),
            pltpu.SemaphoreType.DMA,
            pltpu.SemaphoreType.DMA,
            pltpu.SemaphoreType.DMA,
        ],
    )


def _gath(ct, idx3):
    return _get_gath()(ct, idx3)


_GRID = NP // BM


def _tables_body(ide, cate, idxe, w1, w2, w3, ct):
    ct[0:256, :] = jnp.dot(jnp.maximum(ide[...], 0.0), w1[...],
                           preferred_element_type=jnp.float32)
    ct[256:296, :] = jnp.dot(jnp.maximum(cate[...], 0.0), w2[...],
                             preferred_element_type=jnp.float32)
    ct[296:3360, :] = jnp.dot(jnp.maximum(idxe[...], 0.0), w3[...],
                              preferred_element_type=jnp.float32)


def _node_body(pf, pw, pb, gath, nw, nb, hist, node, dinv):
    pos = jnp.maximum(
        jnp.dot(pf[...], pw[...], preferred_element_type=jnp.float32)
        + pb[...], 0.0)
    node[...] = jnp.maximum(
        jnp.dot(pos, nw[...], preferred_element_type=jnp.float32)
        + gath[...] + nb[...], 0.0)
    deg = 1.0 + hist[0, :, 0:1] + hist[1, :, 0:1]
    dinv[...] = jnp.broadcast_to(lax.rsqrt(deg), (BM, 8))


def _mm_scale_body(x, w, dinv, o):
    h = jnp.dot(x[...], w[...], preferred_element_type=jnp.float32)
    o[...] = h * dinv[:, 0:1]


def _gcn_post_body(p, hs, dinv, b, o):
    q = p[0] + p[1] + hs[...]
    o[...] = jnp.maximum(q * dinv[:, 0:1] + b[...], 0.0)


def _gin_body(gcn, p, w1, b1, w2, b2, o):
    hg = gcn[...] + p[0] + p[1]
    y = jnp.maximum(
        jnp.dot(hg, w1[...], preferred_element_type=jnp.float32) + b1[...],
        0.0)
    o[...] = jnp.maximum(
        jnp.dot(y, w2[...], preferred_element_type=jnp.float32) + b2[...],
        0.0)


def _pool_body(x, bb, o):
    @pl.when(pl.program_id(0) == 0)
    def _init():
        o[...] = jnp.full((G, D), -jnp.inf, jnp.float32)
    xv = x[...]
    b = bb[:, 0:1]
    rows = []
    for g in range(G):
        v = jnp.max(jnp.where(b == g, xv, -jnp.inf), axis=0, keepdims=True)
        rows.append(v)
    o[...] = jnp.maximum(o[...], jnp.concatenate(rows, axis=0))


def _head_body(ge, agW, agb, pdW, pdb, pfW, pfb, idW, idb, ifW, ifb,
               cdW, cdb, cfW, cfb, pos_o, id_o, cat_o):
    def mm(a, b):
        return jnp.dot(a, b[...], preferred_element_type=jnp.float32)

    latent = mm(ge[...], agW) + agb[...]
    t = jnp.maximum(mm(latent, pdW) + pdb[...], 0.0)
    z = mm(t, pfW) + pfb[...]
    pos_o[...] = 1.0 / (1.0 + jnp.exp(-z))
    t = jnp.maximum(mm(latent, idW) + idb[...], 0.0)
    z = mm(t, ifW) + ifb[...]
    e = jnp.exp(z - jnp.max(z, axis=-1, keepdims=True))
    id_o[...] = e / jnp.sum(e, axis=-1, keepdims=True)
    t = jnp.maximum(mm(latent, cdW) + cdb[...], 0.0)
    z = mm(t, cfW) + cfb[...]
    e = jnp.exp(z - jnp.max(z, axis=-1, keepdims=True))
    cat_o[...] = e / jnp.sum(e, axis=-1, keepdims=True)


def _row_spec(shape):
    return pl.BlockSpec(shape, lambda i: (i,) + (0,) * (len(shape) - 1))


def _fix_spec(shape):
    return pl.BlockSpec(shape, lambda i: (0,) * len(shape))


def _tables(ide, cate, idxe, w1, w2, w3):
    return pl.pallas_call(
        _tables_body,
        out_shape=jax.ShapeDtypeStruct((3360, D), jnp.float32),
    )(ide, cate, idxe, w1, w2, w3)


def _node(pf, pw, pb, gath, nw, nb, hist):
    return pl.pallas_call(
        _node_body,
        grid=(_GRID,),
        in_specs=[
            _row_spec((BM, 8)), _fix_spec((8, D)), _fix_spec((1, D)),
            _row_spec((BM, D)), _fix_spec((D, D)), _fix_spec((1, D)),
            pl.BlockSpec((NC, BM, D), lambda i: (0, i, 0)),
        ],
        out_specs=[_row_spec((BM, D)), _row_spec((BM, 8))],
        out_shape=[jax.ShapeDtypeStruct((NP, D), jnp.float32),
                   jax.ShapeDtypeStruct((NP, 8), jnp.float32)],
    )(pf, pw, pb, gath, nw, nb, hist)


def _mm_scale(x, w, dinv):
    return pl.pallas_call(
        _mm_scale_body,
        grid=(_GRID,),
        in_specs=[_row_spec((BM, D)), _fix_spec((D, D)), _row_spec((BM, 8))],
        out_specs=_row_spec((BM, D)),
        out_shape=jax.ShapeDtypeStruct((NP, D), jnp.float32),
    )(x, w, dinv)


def _gcn_post(p, hs, dinv, b):
    return pl.pallas_call(
        _gcn_post_body,
        grid=(_GRID,),
        in_specs=[
            pl.BlockSpec((NC, BM, D), lambda i: (0, i, 0)),
            _row_spec((BM, D)), _row_spec((BM, 8)), _fix_spec((1, D)),
        ],
        out_specs=_row_spec((BM, D)),
        out_shape=jax.ShapeDtypeStruct((NP, D), jnp.float32),
    )(p, hs, dinv, b)


def _gin(gcn, p, w1, b1, w2, b2):
    return pl.pallas_call(
        _gin_body,
        grid=(_GRID,),
        in_specs=[
            _row_spec((BM, D)),
            pl.BlockSpec((NC, BM, D), lambda i: (0, i, 0)),
            _fix_spec((D, D)), _fix_spec((1, D)),
            _fix_spec((D, D)), _fix_spec((1, D)),
        ],
        out_specs=_row_spec((BM, D)),
        out_shape=jax.ShapeDtypeStruct((NP, D), jnp.float32),
    )(gcn, p, w1, b1, w2, b2)


def _pool(x, bb):
    return pl.pallas_call(
        _pool_body,
        grid=(_GRID,),
        in_specs=[_row_spec((BM, D)), _row_spec((BM, 8))],
        out_specs=_fix_spec((G, D)),
        out_shape=jax.ShapeDtypeStruct((G, D), jnp.float32),
    )(x, bb)


def _head(ge, agW, agb, pdW, pdb, pfW, pfb, idW, idb, ifW, ifb,
          cdW, cdb, cfW, cfb):
    return pl.pallas_call(
        _head_body,
        out_shape=[jax.ShapeDtypeStruct((G, 8), jnp.float32),
                   jax.ShapeDtypeStruct((G, 256), jnp.float32),
                   jax.ShapeDtypeStruct((G, D), jnp.float32)],
    )(ge, agW, agb, pdW, pdb, pfW, pfb, idW, idb, ifW, ifb,
      cdW, cdb, cfW, cfb)


def kernel(position_feature, id_feature, category_feature, idx_feature,
           edge_index, edge_attr, batch, pW, pb, id_emb, cat_emb, idx_emb,
           neW, neb, tW, tb, gW1, gb1, gW2, gb2, agW, agb, pdW, pdb, pfW,
           pfb, idW, idb, ifW, ifb, cdW, cdb, cfW, cfb):
    i32 = jnp.int32
    src = edge_index[0].astype(i32)
    dst = edge_index[1].astype(i32)
    act = edge_attr.astype(i32) == 1
    pad_e = (0, EP - E)
    trash_p = DUMMY + (jnp.arange(EP, dtype=i32) % (NP - N))
    act_t = jnp.pad(act, pad_e)
    act_s = jnp.pad(~act, pad_e)
    dst_p = jnp.pad(dst, pad_e)
    src_p = jnp.pad(src, pad_e).reshape(NW, NBE, KB)
    src_t = src_p
    src_s = src_p
    dst_t = jnp.where(act_t, dst_p, trash_p).reshape(NW, NBE, KB)
    dst_s = jnp.where(act_s, dst_p, trash_p).reshape(NW, NBE, KB)

    idx3 = jnp.stack([id_feature.astype(i32),
                      category_feature.astype(i32) + 256,
                      idx_feature.astype(i32) + 296])
    idx3 = jnp.pad(idx3, ((0, 0), (0, NP - N)))
    idx3 = idx3.reshape(3, NW, NBN, KGN).transpose(1, 0, 2, 3)

    bb8 = jnp.broadcast_to(
        jnp.pad(batch.astype(i32), (0, NP - N), constant_values=G)[:, None],
        (NP, 8))
    pf8 = jnp.pad(position_feature, ((0, NP - N), (0, 5)))
    pW8 = jnp.pad(pW, ((0, 5), (0, 0)))
    cat_emb_p = jnp.pad(cat_emb, ((0, 6), (0, 0)))
    idx_emb_p = jnp.pad(idx_emb, ((0, 4), (0, 0)))
    pfW8 = jnp.pad(pfW, ((0, 0), (0, 5)))
    pfb8 = jnp.pad(pfb, (0, 5))
    cfW128 = jnp.pad(cfW, ((0, 0), (0, D - 34)))
    cfb128 = jnp.pad(cfb, (0, D - 34), constant_values=NEG)

    histp = _spmm(jnp.ones((NP, D), jnp.float32), src_t, dst_t)
    ct = _tables(id_emb, cat_emb_p, idx_emb_p,
                 neW[128:256], neW[256:384], neW[384:512])
    gath = _gath(ct, idx3)
    node, dinv8 = _node(pf8, pW8, pb[None], gath, neW[0:128], neb[None],
                        histp)

    pools = [_pool(node, bb8)]
    x = node
    for i in range(NLAYER):
        hs = _mm_scale(x, tW[i], dinv8)
        p = _spmm(hs, src_t, dst_t)
        gcn = _gcn_post(p, hs, dinv8, tb[i][None])
        q = _spmm(gcn, src_s, dst_s)
        nt = _gin(gcn, q, gW1[i], gb1[i][None], gW2[i], gb2[i][None])
        pools.append(_pool(nt, bb8))
        x = nt

    ge = jnp.concatenate(pools, axis=1)
    pos8, id_out, cat128 = _head(
        ge, agW, agb[None], pdW, pdb[None], pfW8, pfb8[None],
        idW, idb[None], ifW, ifb[None], cdW, cdb[None], cfW128, cfb128[None])
    return pos8[:, :3], id_out, cat128[:, :34]


# speedup vs baseline: 12.7025x; 1.0420x over previous
import functools

import jax
import jax.numpy as jnp
from jax import lax
from jax.experimental import pallas as pl
from jax.experimental.pallas import tpu as pltpu
from jax.experimental.pallas import tpu_sc as plsc

N = 10000
NP = 10240
D = 128
E = 320000
G = 16
NLAYER = 3
NC = 2
NS = 16
NW = NC * NS
KB = 128
NBE = 80
CHB = 8
NCH = NBE // CHB
EBP = NBE * KB
EP = NW * EBP
NBUF = 2
RACC = 11264
ROWS_T = RACC // NS
ZB = 64
NODE_T = NP // NW
KGN = 80
NBN = NODE_T // KGN
DUMMY = N
BM = 1024
NEG = -1e30


def _zero_rows(buf, nrow, width):
    def zrow(r, carry):
        for jj in range(width // 16):
            buf[r, pl.ds(jj * 16, 16)] = jnp.zeros((16,), jnp.float32)
        return carry
    lax.fori_loop(0, nrow, zrow, 0)


def _spmm_body(x_hbm, srcs_hbm, dsts_hbm, out_hbm,
               sa, sb, da, db, r0, r1, acc,
               ia, ib, g0, g1, s0, s1):
    c = lax.axis_index("c")
    s = lax.axis_index("s")
    wid = s * NC + c
    sbank = (sa, sb)
    dbank = (da, db)
    isems = (ia, ib)
    bufs = (r0, r1)
    gsems = (g0, g1)
    ssems = (s0, s1)

    _zero_rows(r0, KB, D)
    def zcp(t, carry):
        pltpu.sync_copy(r0.at[pl.ds(0, ZB)],
                        acc.at[pl.ds(s * ROWS_T + t * ZB, ZB)])
        return carry
    lax.fori_loop(0, ROWS_T // ZB, zcp, 0)
    plsc.subcore_barrier()

    def fire_idx(k, bank):
        pltpu.async_copy(srcs_hbm.at[wid, pl.ds(k * CHB, CHB)],
                         sbank[bank], isems[bank])
        pltpu.async_copy(dsts_hbm.at[wid, pl.ds(k * CHB, CHB)],
                         dbank[bank], isems[bank])

    def wait_idx(bank):
        pltpu.make_async_copy(srcs_hbm.at[wid, pl.ds(0, CHB)],
                              sbank[bank], isems[bank]).wait()
        pltpu.make_async_copy(dsts_hbm.at[wid, pl.ds(0, CHB)],
                              dbank[bank], isems[bank]).wait()

    fire_idx(0, 0)

    def chunk_pair(cp, carry):
        for half in range(2):
            k = cp * 2 + half
            sbk = sbank[half]
            dbk = dbank[half]
            wait_idx(half)
            @pl.when(k + 1 < NCH)
            def _next_idx():
                fire_idx(k + 1, 1 - half)
            pltpu.async_copy(x_hbm.at[sbk.at[0]], bufs[0], gsems[0])
            pltpu.async_copy(x_hbm.at[sbk.at[1]], bufs[1], gsems[1])

            def ring(jl, rcarry):
                for b in range(NBUF):
                    jloc = jl * NBUF + b
                    pltpu.make_async_copy(x_hbm.at[sbk.at[jloc]], bufs[b],
                                          gsems[b]).wait()
                    pltpu.async_copy(bufs[b], acc.at[dbk.at[jloc]],
                                     ssems[b], add=True)
                for b in range(NBUF):
                    jloc = jl * NBUF + b
                    pltpu.make_async_copy(bufs[b], acc.at[dbk.at[jloc]],
                                          ssems[b]).wait()
                    @pl.when(jloc + NBUF < CHB)
                    def _prefetch():
                        pltpu.async_copy(x_hbm.at[sbk.at[jloc + NBUF]],
                                         bufs[b], gsems[b])
                return rcarry
            lax.fori_loop(0, CHB // NBUF, ring, 0)
        return carry
    lax.fori_loop(0, NCH // 2, chunk_pair, 0)
    plsc.subcore_barrier()
    pltpu.sync_copy(acc.at[pl.ds(s * ROWS_T, ROWS_T)],
                    out_hbm.at[c, pl.ds(s * ROWS_T, ROWS_T)])


@functools.cache
def _get_spmm():
    return pl.kernel(
        _spmm_body,
        out_type=jax.ShapeDtypeStruct((NC, RACC, D), jnp.float32),
        mesh=plsc.VectorSubcoreMesh(core_axis_name="c", subcore_axis_name="s"),
        scratch_types=[
            pltpu.VMEM((CHB, KB), jnp.int32),
            pltpu.VMEM((CHB, KB), jnp.int32),
            pltpu.VMEM((CHB, KB), jnp.int32),
            pltpu.VMEM((CHB, KB), jnp.int32),
            pltpu.VMEM((KB, D), jnp.float32),
            pltpu.VMEM((KB, D), jnp.float32),
            pltpu.VMEM_SHARED((RACC, D), jnp.float32),
            pltpu.SemaphoreType.DMA,
            pltpu.SemaphoreType.DMA,
            pltpu.SemaphoreType.DMA,
            pltpu.SemaphoreType.DMA,
            pltpu.SemaphoreType.DMA,
            pltpu.SemaphoreType.DMA,
        ],
    )


def _spmm(x, srcs, dsts):
    return _get_spmm()(x, srcs, dsts)


def _gath_body(ct_hbm, idx_hbm, out_hbm, idx_v, r0, r1, r2, sem0, sem1, sem2):
    c = lax.axis_index("c")
    s = lax.axis_index("s")
    wid = s * NC + c

    pltpu.sync_copy(idx_hbm.at[wid], idx_v)
    for j in range(NBN):
        cp0 = pltpu.async_copy(ct_hbm.at[idx_v.at[0, j]], r0, sem0)
        cp1 = pltpu.async_copy(ct_hbm.at[idx_v.at[1, j]], r1, sem1)
        cp2 = pltpu.async_copy(ct_hbm.at[idx_v.at[2, j]], r2, sem2)
        cp0.wait()
        cp1.wait()
        cp2.wait()
        def addrow(r, carry):
            for jj in range(D // 16):
                sl = pl.ds(jj * 16, 16)
                r0[r, sl] = r0[r, sl] + r1[r, sl] + r2[r, sl]
            return carry
        lax.fori_loop(0, KGN, addrow, 0)
        pltpu.sync_copy(r0, out_hbm.at[pl.ds(wid * NODE_T + j * KGN, KGN)])


@functools.cache
def _get_gath():
    return pl.kernel(
        _gath_body,
        out_type=jax.ShapeDtypeStruct((NP, D), jnp.float32),
        mesh=plsc.VectorSubcoreMesh(core_axis_name="c", subcore_axis_name="s"),
        scratch_types=[
            pltpu.VMEM((3, NBN, KGN), jnp.int32),
            pltpu.VMEM((KGN, D), jnp.float32),
            pltpu.VMEM((KGN, D), jnp.float32),
            pltpu.VMEM((KGN, D), jnp.float32),
            pltpu.SemaphoreType.DMA,
            pltpu.SemaphoreType.DMA,
            pltpu.SemaphoreType.DMA,
        ],
    )


def _gath(ct, idx3):
    return _get_gath()(ct, idx3)


_GRID = NP // BM


def _tables_body(ide, cate, idxe, w1, w2, w3, ct):
    ct[0:256, :] = jnp.dot(jnp.maximum(ide[...], 0.0), w1[...],
                           preferred_element_type=jnp.float32)
    ct[256:296, :] = jnp.dot(jnp.maximum(cate[...], 0.0), w2[...],
                             preferred_element_type=jnp.float32)
    ct[296:3360, :] = jnp.dot(jnp.maximum(idxe[...], 0.0), w3[...],
                              preferred_element_type=jnp.float32)


def _node_body(pf, pw, pb, gath, nw, nb, hist, node, dinv):
    pos = jnp.maximum(
        jnp.dot(pf[...], pw[...], preferred_element_type=jnp.float32)
        + pb[...], 0.0)
    node[...] = jnp.maximum(
        jnp.dot(pos, nw[...], preferred_element_type=jnp.float32)
        + gath[...] + nb[...], 0.0)
    deg = 1.0 + hist[0, :, 0:1] + hist[1, :, 0:1]
    dinv[...] = jnp.broadcast_to(lax.rsqrt(deg), (BM, 8))


def _mm_scale_body(x, w, dinv, o):
    h = jnp.dot(x[...], w[...], preferred_element_type=jnp.float32)
    o[...] = h * dinv[:, 0:1]


def _gcn_post_body(p, hs, dinv, b, o):
    q = p[0] + p[1] + hs[...]
    o[...] = jnp.maximum(q * dinv[:, 0:1] + b[...], 0.0)


def _gin_body(gcn, p, w1, b1, w2, b2, o):
    hg = gcn[...] + p[0] + p[1]
    y = jnp.maximum(
        jnp.dot(hg, w1[...], preferred_element_type=jnp.float32) + b1[...],
        0.0)
    o[...] = jnp.maximum(
        jnp.dot(y, w2[...], preferred_element_type=jnp.float32) + b2[...],
        0.0)


def _pool_body(x, bb, o):
    @pl.when(pl.program_id(0) == 0)
    def _init():
        o[...] = jnp.full((G, D), -jnp.inf, jnp.float32)
    xv = x[...]
    b = bb[:, 0:1]
    rows = []
    for g in range(G):
        v = jnp.max(jnp.where(b == g, xv, -jnp.inf), axis=0, keepdims=True)
        rows.append(v)
    o[...] = jnp.maximum(o[...], jnp.concatenate(rows, axis=0))


def _head_body(ge, agW, agb, pdW, pdb, pfW, pfb, idW, idb, ifW, ifb,
               cdW, cdb, cfW, cfb, pos_o, id_o, cat_o):
    def mm(a, b):
        return jnp.dot(a, b[...], preferred_element_type=jnp.float32)

    latent = mm(ge[...], agW) + agb[...]
    t = jnp.maximum(mm(latent, pdW) + pdb[...], 0.0)
    z = mm(t, pfW) + pfb[...]
    pos_o[...] = 1.0 / (1.0 + jnp.exp(-z))
    t = jnp.maximum(mm(latent, idW) + idb[...], 0.0)
    z = mm(t, ifW) + ifb[...]
    e = jnp.exp(z - jnp.max(z, axis=-1, keepdims=True))
    id_o[...] = e / jnp.sum(e, axis=-1, keepdims=True)
    t = jnp.maximum(mm(latent, cdW) + cdb[...], 0.0)
    z = mm(t, cfW) + cfb[...]
    e = jnp.exp(z - jnp.max(z, axis=-1, keepdims=True))
    cat_o[...] = e / jnp.sum(e, axis=-1, keepdims=True)


def _row_spec(shape):
    return pl.BlockSpec(shape, lambda i: (i,) + (0,) * (len(shape) - 1))


def _fix_spec(shape):
    return pl.BlockSpec(shape, lambda i: (0,) * len(shape))


def _tables(ide, cate, idxe, w1, w2, w3):
    return pl.pallas_call(
        _tables_body,
        out_shape=jax.ShapeDtypeStruct((3360, D), jnp.float32),
    )(ide, cate, idxe, w1, w2, w3)


def _node(pf, pw, pb, gath, nw, nb, hist):
    return pl.pallas_call(
        _node_body,
        grid=(_GRID,),
        in_specs=[
            _row_spec((BM, 8)), _fix_spec((8, D)), _fix_spec((1, D)),
            _row_spec((BM, D)), _fix_spec((D, D)), _fix_spec((1, D)),
            pl.BlockSpec((NC, BM, D), lambda i: (0, i, 0)),
        ],
        out_specs=[_row_spec((BM, D)), _row_spec((BM, 8))],
        out_shape=[jax.ShapeDtypeStruct((NP, D), jnp.float32),
                   jax.ShapeDtypeStruct((NP, 8), jnp.float32)],
    )(pf, pw, pb, gath, nw, nb, hist)


def _mm_scale(x, w, dinv):
    return pl.pallas_call(
        _mm_scale_body,
        grid=(_GRID,),
        in_specs=[_row_spec((BM, D)), _fix_spec((D, D)), _row_spec((BM, 8))],
        out_specs=_row_spec((BM, D)),
        out_shape=jax.ShapeDtypeStruct((NP, D), jnp.float32),
    )(x, w, dinv)


def _gcn_post(p, hs, dinv, b):
    return pl.pallas_call(
        _gcn_post_body,
        grid=(_GRID,),
        in_specs=[
            pl.BlockSpec((NC, BM, D), lambda i: (0, i, 0)),
            _row_spec((BM, D)), _row_spec((BM, 8)), _fix_spec((1, D)),
        ],
        out_specs=_row_spec((BM, D)),
        out_shape=jax.ShapeDtypeStruct((NP, D), jnp.float32),
    )(p, hs, dinv, b)


def _gin(gcn, p, w1, b1, w2, b2):
    return pl.pallas_call(
        _gin_body,
        grid=(_GRID,),
        in_specs=[
            _row_spec((BM, D)),
            pl.BlockSpec((NC, BM, D), lambda i: (0, i, 0)),
            _fix_spec((D, D)), _fix_spec((1, D)),
            _fix_spec((D, D)), _fix_spec((1, D)),
        ],
        out_specs=_row_spec((BM, D)),
        out_shape=jax.ShapeDtypeStruct((NP, D), jnp.float32),
    )(gcn, p, w1, b1, w2, b2)


def _pool(x, bb):
    return pl.pallas_call(
        _pool_body,
        grid=(_GRID,),
        in_specs=[_row_spec((BM, D)), _row_spec((BM, 8))],
        out_specs=_fix_spec((G, D)),
        out_shape=jax.ShapeDtypeStruct((G, D), jnp.float32),
    )(x, bb)


def _head(ge, agW, agb, pdW, pdb, pfW, pfb, idW, idb, ifW, ifb,
          cdW, cdb, cfW, cfb):
    return pl.pallas_call(
        _head_body,
        out_shape=[jax.ShapeDtypeStruct((G, 8), jnp.float32),
                   jax.ShapeDtypeStruct((G, 256), jnp.float32),
                   jax.ShapeDtypeStruct((G, D), jnp.float32)],
    )(ge, agW, agb, pdW, pdb, pfW, pfb, idW, idb, ifW, ifb,
      cdW, cdb, cfW, cfb)


def kernel(position_feature, id_feature, category_feature, idx_feature,
           edge_index, edge_attr, batch, pW, pb, id_emb, cat_emb, idx_emb,
           neW, neb, tW, tb, gW1, gb1, gW2, gb2, agW, agb, pdW, pdb, pfW,
           pfb, idW, idb, ifW, ifb, cdW, cdb, cfW, cfb):
    i32 = jnp.int32
    src = edge_index[0].astype(i32)
    dst = edge_index[1].astype(i32)
    act = edge_attr.astype(i32) == 1
    pad_e = (0, EP - E)
    trash_p = DUMMY + (jnp.arange(EP, dtype=i32) % (RACC - N))
    act_t = jnp.pad(act, pad_e)
    act_s = jnp.pad(~act, pad_e)
    dst_p = jnp.pad(dst, pad_e)
    src_p = jnp.pad(src, pad_e).reshape(NW, NBE, KB)
    src_t = src_p
    src_s = src_p
    dst_t = jnp.where(act_t, dst_p, trash_p).reshape(NW, NBE, KB)
    dst_s = jnp.where(act_s, dst_p, trash_p).reshape(NW, NBE, KB)

    idx3 = jnp.stack([id_feature.astype(i32),
                      category_feature.astype(i32) + 256,
                      idx_feature.astype(i32) + 296])
    idx3 = jnp.pad(idx3, ((0, 0), (0, NP - N)))
    idx3 = idx3.reshape(3, NW, NBN, KGN).transpose(1, 0, 2, 3)

    bb8 = jnp.broadcast_to(
        jnp.pad(batch.astype(i32), (0, NP - N), constant_values=G)[:, None],
        (NP, 8))
    pf8 = jnp.pad(position_feature, ((0, NP - N), (0, 5)))
    pW8 = jnp.pad(pW, ((0, 5), (0, 0)))
    cat_emb_p = jnp.pad(cat_emb, ((0, 6), (0, 0)))
    idx_emb_p = jnp.pad(idx_emb, ((0, 4), (0, 0)))
    pfW8 = jnp.pad(pfW, ((0, 0), (0, 5)))
    pfb8 = jnp.pad(pfb, (0, 5))
    cfW128 = jnp.pad(cfW, ((0, 0), (0, D - 34)))
    cfb128 = jnp.pad(cfb, (0, D - 34), constant_values=NEG)

    histp = _spmm(jnp.ones((NP, D), jnp.float32), src_t, dst_t)
    ct = _tables(id_emb, cat_emb_p, idx_emb_p,
                 neW[128:256], neW[256:384], neW[384:512])
    gath = _gath(ct, idx3)
    node, dinv8 = _node(pf8, pW8, pb[None], gath, neW[0:128], neb[None],
                        histp)

    pools = [_pool(node, bb8)]
    x = node
    for i in range(NLAYER):
        hs = _mm_scale(x, tW[i], dinv8)
        p = _spmm(hs, src_t, dst_t)
        gcn = _gcn_post(p, hs, dinv8, tb[i][None])
        q = _spmm(gcn, src_s, dst_s)
        nt = _gin(gcn, q, gW1[i], gb1[i][None], gW2[i], gb2[i][None])
        pools.append(_pool(nt, bb8))
        x = nt

    ge = jnp.concatenate(pools, axis=1)
    pos8, id_out, cat128 = _head(
        ge, agW, agb[None], pdW, pdb[None], pfW8, pfb8[None],
        idW, idb[None], ifW, ifb[None], cdW, cdb[None], cfW128, cfb128[None])
    return pos8[:, :3], id_out, cat128[:, :34]


# speedup vs baseline: 12.9253x; 1.0175x over previous
import functools

import jax
import jax.numpy as jnp
from jax import lax
from jax.experimental import pallas as pl
from jax.experimental.pallas import tpu as pltpu
from jax.experimental.pallas import tpu_sc as plsc

N = 10000
NP = 10240
D = 128
E = 320000
G = 16
NLAYER = 3
NC = 2
NS = 16
NW = NC * NS
KB = 128
NBE = 80
CHB = 8
NCH = NBE // CHB
EBP = NBE * KB
EP = NW * EBP
NBUF = 2
RACC = 11264
ROWS_T = RACC // NS
ZB = 64
NODE_T = NP // NW
KGN = 64
NBN = NODE_T // KGN
HKB = 64
HNB = EBP // HKB
HCH = 8
HNCH = HNB // HCH
DUMMY = N
BM = 1024
NEG = -1e30


def _zero_rows(buf, nrow, width):
    def zrow(r, carry):
        for jj in range(width // 16):
            buf[r, pl.ds(jj * 16, 16)] = jnp.zeros((16,), jnp.float32)
        return carry
    lax.fori_loop(0, nrow, zrow, 0)


def _spmm_body(x_hbm, srcs_hbm, dsts_hbm, out_hbm,
               sa, sb, da, db, r0, r1, acc,
               ia, ib, g0, g1, s0, s1):
    c = lax.axis_index("c")
    s = lax.axis_index("s")
    wid = s * NC + c
    sbank = (sa, sb)
    dbank = (da, db)
    isems = (ia, ib)
    bufs = (r0, r1)
    gsems = (g0, g1)
    ssems = (s0, s1)

    _zero_rows(r0, KB, D)
    def zcp(t, carry):
        pltpu.sync_copy(r0.at[pl.ds(0, ZB)],
                        acc.at[pl.ds(s * ROWS_T + t * ZB, ZB)])
        return carry
    lax.fori_loop(0, ROWS_T // ZB, zcp, 0)
    plsc.subcore_barrier()

    def fire_idx(k, bank):
        pltpu.async_copy(srcs_hbm.at[wid, pl.ds(k * CHB, CHB)],
                         sbank[bank], isems[bank])
        pltpu.async_copy(dsts_hbm.at[wid, pl.ds(k * CHB, CHB)],
                         dbank[bank], isems[bank])

    def wait_idx(bank):
        pltpu.make_async_copy(srcs_hbm.at[wid, pl.ds(0, CHB)],
                              sbank[bank], isems[bank]).wait()
        pltpu.make_async_copy(dsts_hbm.at[wid, pl.ds(0, CHB)],
                              dbank[bank], isems[bank]).wait()

    fire_idx(0, 0)

    def chunk_pair(cp, carry):
        for half in range(2):
            k = cp * 2 + half
            sbk = sbank[half]
            dbk = dbank[half]
            wait_idx(half)
            @pl.when(k + 1 < NCH)
            def _next_idx():
                fire_idx(k + 1, 1 - half)
            pltpu.async_copy(x_hbm.at[sbk.at[0]], bufs[0], gsems[0])
            pltpu.async_copy(x_hbm.at[sbk.at[1]], bufs[1], gsems[1])

            def ring(jl, rcarry):
                for b in range(NBUF):
                    jloc = jl * NBUF + b
                    pltpu.make_async_copy(x_hbm.at[sbk.at[jloc]], bufs[b],
                                          gsems[b]).wait()
                    pltpu.async_copy(bufs[b], acc.at[dbk.at[jloc]],
                                     ssems[b], add=True)
                for b in range(NBUF):
                    jloc = jl * NBUF + b
                    pltpu.make_async_copy(bufs[b], acc.at[dbk.at[jloc]],
                                          ssems[b]).wait()
                    @pl.when(jloc + NBUF < CHB)
                    def _prefetch():
                        pltpu.async_copy(x_hbm.at[sbk.at[jloc + NBUF]],
                                         bufs[b], gsems[b])
                return rcarry
            lax.fori_loop(0, CHB // NBUF, ring, 0)
        return carry
    lax.fori_loop(0, NCH // 2, chunk_pair, 0)
    plsc.subcore_barrier()
    pltpu.sync_copy(acc.at[pl.ds(s * ROWS_T, ROWS_T)],
                    out_hbm.at[c, pl.ds(s * ROWS_T, ROWS_T)])


@functools.cache
def _get_spmm():
    return pl.kernel(
        _spmm_body,
        out_type=jax.ShapeDtypeStruct((NC, RACC, D), jnp.float32),
        mesh=plsc.VectorSubcoreMesh(core_axis_name="c", subcore_axis_name="s"),
        scratch_types=[
            pltpu.VMEM((CHB, KB), jnp.int32),
            pltpu.VMEM((CHB, KB), jnp.int32),
            pltpu.VMEM((CHB, KB), jnp.int32),
            pltpu.VMEM((CHB, KB), jnp.int32),
            pltpu.VMEM((KB, D), jnp.float32),
            pltpu.VMEM((KB, D), jnp.float32),
            pltpu.VMEM_SHARED((RACC, D), jnp.float32),
            pltpu.SemaphoreType.DMA,
            pltpu.SemaphoreType.DMA,
            pltpu.SemaphoreType.DMA,
            pltpu.SemaphoreType.DMA,
            pltpu.SemaphoreType.DMA,
            pltpu.SemaphoreType.DMA,
        ],
    )


def _spmm(x, srcs, dsts):
    return _get_spmm()(x, srcs, dsts)


def _gath_hist_body(ct_hbm, idx_hbm, dsth_hbm, gout_hbm, hout_hbm,
                    idx_v, r0, r1, r2, da, db, ones_v, acc,
                    sem0, sem1, sem2, ia, ib, ssem):
    c = lax.axis_index("c")
    s = lax.axis_index("s")
    wid = s * NC + c
    dbank = (da, db)
    isems = (ia, ib)

    _zero_rows(r0, HKB, D)
    def zcp(t, carry):
        pltpu.sync_copy(r0.at[pl.ds(0, ZB)],
                        acc.at[pl.ds(s * ROWS_T + t * ZB, ZB)])
        return carry
    lax.fori_loop(0, ROWS_T // ZB, zcp, 0)
    def orow(r, carry):
        for jj in range(D // 16):
            ones_v[r, pl.ds(jj * 16, 16)] = jnp.ones((16,), jnp.float32)
        return carry
    lax.fori_loop(0, HKB, orow, 0)
    plsc.subcore_barrier()

    def fire_hidx(k, bank):
        pltpu.async_copy(dsth_hbm.at[wid, pl.ds(k * HCH, HCH)],
                         dbank[bank], isems[bank])

    def wait_hidx(bank):
        pltpu.make_async_copy(dsth_hbm.at[wid, pl.ds(0, HCH)],
                              dbank[bank], isems[bank]).wait()

    fire_hidx(0, 0)
    def hchunk_pair(cp, carry):
        for half in range(2):
            k = cp * 2 + half
            dbk = dbank[half]
            wait_hidx(half)
            @pl.when(k + 1 < HNCH)
            def _next_idx():
                fire_hidx(k + 1, 1 - half)
            for j in range(HCH):
                pltpu.async_copy(ones_v, acc.at[dbk.at[j]], ssem, add=True)
            for j in range(HCH):
                pltpu.make_async_copy(ones_v, acc.at[dbk.at[j]], ssem).wait()
        return carry
    lax.fori_loop(0, HNCH // 2, hchunk_pair, 0)

    pltpu.sync_copy(idx_hbm.at[wid], idx_v)
    for j in range(NBN):
        cp0 = pltpu.async_copy(ct_hbm.at[idx_v.at[0, j]], r0, sem0)
        cp1 = pltpu.async_copy(ct_hbm.at[idx_v.at[1, j]], r1, sem1)
        cp2 = pltpu.async_copy(ct_hbm.at[idx_v.at[2, j]], r2, sem2)
        cp0.wait()
        cp1.wait()
        cp2.wait()
        def addrow(r, carry):
            for jj in range(D // 16):
                sl = pl.ds(jj * 16, 16)
                r0[r, sl] = r0[r, sl] + r1[r, sl] + r2[r, sl]
            return carry
        lax.fori_loop(0, KGN, addrow, 0)
        pltpu.sync_copy(r0, gout_hbm.at[pl.ds(wid * NODE_T + j * KGN, KGN)])

    plsc.subcore_barrier()
    pltpu.sync_copy(acc.at[pl.ds(s * ROWS_T, ROWS_T)],
                    hout_hbm.at[c, pl.ds(s * ROWS_T, ROWS_T)])


@functools.cache
def _get_gath():
    return pl.kernel(
        _gath_hist_body,
        out_type=[jax.ShapeDtypeStruct((NP, D), jnp.float32),
                  jax.ShapeDtypeStruct((NC, RACC, D), jnp.float32)],
        mesh=plsc.VectorSubcoreMesh(core_axis_name="c", subcore_axis_name="s"),
        scratch_types=[
            pltpu.VMEM((3, NBN, KGN), jnp.int32),
            pltpu.VMEM((KGN, D), jnp.float32),
            pltpu.VMEM((KGN, D), jnp.float32),
            pltpu.VMEM((KGN, D), jnp.float32),
            pltpu.VMEM((HCH, HKB), jnp.int32),
            pltpu.VMEM((HCH, HKB), jnp.int32),
            pltpu.VMEM((HKB, D), jnp.float32),
            pltpu.VMEM_SHARED((RACC, D), jnp.float32),
            pltpu.SemaphoreType.DMA,
            pltpu.SemaphoreType.DMA,
            pltpu.SemaphoreType.DMA,
            pltpu.SemaphoreType.DMA,
            pltpu.SemaphoreType.DMA,
            pltpu.SemaphoreType.DMA,
        ],
    )


def _gath(ct, idx3, dsth):
    return _get_gath()(ct, idx3, dsth)


_GRID = NP // BM


def _tables_body(ide, cate, idxe, w1, w2, w3, ct):
    ct[0:256, :] = jnp.dot(jnp.maximum(ide[...], 0.0), w1[...],
                           preferred_element_type=jnp.float32)
    ct[256:296, :] = jnp.dot(jnp.maximum(cate[...], 0.0), w2[...],
                             preferred_element_type=jnp.float32)
    ct[296:3360, :] = jnp.dot(jnp.maximum(idxe[...], 0.0), w3[...],
                              preferred_element_type=jnp.float32)


def _node_body(pf, pw, pb, gath, nw, nb, hist, node, dinv):
    pos = jnp.maximum(
        jnp.dot(pf[...], pw[...], preferred_element_type=jnp.float32)
        + pb[...], 0.0)
    node[...] = jnp.maximum(
        jnp.dot(pos, nw[...], preferred_element_type=jnp.float32)
        + gath[...] + nb[...], 0.0)
    deg = 1.0 + hist[0, :, 0:1] + hist[1, :, 0:1]
    dinv[...] = jnp.broadcast_to(lax.rsqrt(deg), (BM, 8))


def _mm_scale_body(x, w, dinv, o):
    h = jnp.dot(x[...], w[...], preferred_element_type=jnp.float32)
    o[...] = h * dinv[:, 0:1]


def _gcn_post_body(p, hs, dinv, b, o):
    q = p[0] + p[1] + hs[...]
    o[...] = jnp.maximum(q * dinv[:, 0:1] + b[...], 0.0)


def _gin_body(gcn, p, w1, b1, w2, b2, o):
    hg = gcn[...] + p[0] + p[1]
    y = jnp.maximum(
        jnp.dot(hg, w1[...], preferred_element_type=jnp.float32) + b1[...],
        0.0)
    o[...] = jnp.maximum(
        jnp.dot(y, w2[...], preferred_element_type=jnp.float32) + b2[...],
        0.0)


def _pool_body(x, bb, o):
    @pl.when(pl.program_id(0) == 0)
    def _init():
        o[...] = jnp.full((G, D), -jnp.inf, jnp.float32)
    xv = x[...]
    b = bb[:, 0:1]
    rows = []
    for g in range(G):
        v = jnp.max(jnp.where(b == g, xv, -jnp.inf), axis=0, keepdims=True)
        rows.append(v)
    o[...] = jnp.maximum(o[...], jnp.concatenate(rows, axis=0))


def _head_body(ge, agW, agb, pdW, pdb, pfW, pfb, idW, idb, ifW, ifb,
               cdW, cdb, cfW, cfb, pos_o, id_o, cat_o):
    def mm(a, b):
        return jnp.dot(a, b[...], preferred_element_type=jnp.float32)

    latent = mm(ge[...], agW) + agb[...]
    t = jnp.maximum(mm(latent, pdW) + pdb[...], 0.0)
    z = mm(t, pfW) + pfb[...]
    pos_o[...] = 1.0 / (1.0 + jnp.exp(-z))
    t = jnp.maximum(mm(latent, idW) + idb[...], 0.0)
    z = mm(t, ifW) + ifb[...]
    e = jnp.exp(z - jnp.max(z, axis=-1, keepdims=True))
    id_o[...] = e / jnp.sum(e, axis=-1, keepdims=True)
    t = jnp.maximum(mm(latent, cdW) + cdb[...], 0.0)
    z = mm(t, cfW) + cfb[...]
    e = jnp.exp(z - jnp.max(z, axis=-1, keepdims=True))
    cat_o[...] = e / jnp.sum(e, axis=-1, keepdims=True)


def _row_spec(shape):
    return pl.BlockSpec(shape, lambda i: (i,) + (0,) * (len(shape) - 1))


def _fix_spec(shape):
    return pl.BlockSpec(shape, lambda i: (0,) * len(shape))


def _tables(ide, cate, idxe, w1, w2, w3):
    return pl.pallas_call(
        _tables_body,
        out_shape=jax.ShapeDtypeStruct((3360, D), jnp.float32),
    )(ide, cate, idxe, w1, w2, w3)


def _node(pf, pw, pb, gath, nw, nb, hist):
    return pl.pallas_call(
        _node_body,
        grid=(_GRID,),
        in_specs=[
            _row_spec((BM, 8)), _fix_spec((8, D)), _fix_spec((1, D)),
            _row_spec((BM, D)), _fix_spec((D, D)), _fix_spec((1, D)),
            pl.BlockSpec((NC, BM, D), lambda i: (0, i, 0)),
        ],
        out_specs=[_row_spec((BM, D)), _row_spec((BM, 8))],
        out_shape=[jax.ShapeDtypeStruct((NP, D), jnp.float32),
                   jax.ShapeDtypeStruct((NP, 8), jnp.float32)],
    )(pf, pw, pb, gath, nw, nb, hist)


def _mm_scale(x, w, dinv):
    return pl.pallas_call(
        _mm_scale_body,
        grid=(_GRID,),
        in_specs=[_row_spec((BM, D)), _fix_spec((D, D)), _row_spec((BM, 8))],
        out_specs=_row_spec((BM, D)),
        out_shape=jax.ShapeDtypeStruct((NP, D), jnp.float32),
    )(x, w, dinv)


def _gcn_post(p, hs, dinv, b):
    return pl.pallas_call(
        _gcn_post_body,
        grid=(_GRID,),
        in_specs=[
            pl.BlockSpec((NC, BM, D), lambda i: (0, i, 0)),
            _row_spec((BM, D)), _row_spec((BM, 8)), _fix_spec((1, D)),
        ],
        out_specs=_row_spec((BM, D)),
        out_shape=jax.ShapeDtypeStruct((NP, D), jnp.float32),
    )(p, hs, dinv, b)


def _gin(gcn, p, w1, b1, w2, b2):
    return pl.pallas_call(
        _gin_body,
        grid=(_GRID,),
        in_specs=[
            _row_spec((BM, D)),
            pl.BlockSpec((NC, BM, D), lambda i: (0, i, 0)),
            _fix_spec((D, D)), _fix_spec((1, D)),
            _fix_spec((D, D)), _fix_spec((1, D)),
        ],
        out_specs=_row_spec((BM, D)),
        out_shape=jax.ShapeDtypeStruct((NP, D), jnp.float32),
    )(gcn, p, w1, b1, w2, b2)


def _pool(x, bb):
    return pl.pallas_call(
        _pool_body,
        grid=(_GRID,),
        in_specs=[_row_spec((BM, D)), _row_spec((BM, 8))],
        out_specs=_fix_spec((G, D)),
        out_shape=jax.ShapeDtypeStruct((G, D), jnp.float32),
    )(x, bb)


def _head(ge, agW, agb, pdW, pdb, pfW, pfb, idW, idb, ifW, ifb,
          cdW, cdb, cfW, cfb):
    return pl.pallas_call(
        _head_body,
        out_shape=[jax.ShapeDtypeStruct((G, 8), jnp.float32),
                   jax.ShapeDtypeStruct((G, 256), jnp.float32),
                   jax.ShapeDtypeStruct((G, D), jnp.float32)],
    )(ge, agW, agb, pdW, pdb, pfW, pfb, idW, idb, ifW, ifb,
      cdW, cdb, cfW, cfb)


def kernel(position_feature, id_feature, category_feature, idx_feature,
           edge_index, edge_attr, batch, pW, pb, id_emb, cat_emb, idx_emb,
           neW, neb, tW, tb, gW1, gb1, gW2, gb2, agW, agb, pdW, pdb, pfW,
           pfb, idW, idb, ifW, ifb, cdW, cdb, cfW, cfb):
    i32 = jnp.int32
    src = edge_index[0].astype(i32)
    dst = edge_index[1].astype(i32)
    act = edge_attr.astype(i32) == 1
    pad_e = (0, EP - E)
    trash_p = DUMMY + (jnp.arange(EP, dtype=i32) % (RACC - N))
    act_t = jnp.pad(act, pad_e)
    act_s = jnp.pad(~act, pad_e)
    dst_p = jnp.pad(dst, pad_e)
    src_p = jnp.pad(src, pad_e).reshape(NW, NBE, KB)
    src_t = src_p
    src_s = src_p
    dst_t1 = jnp.where(act_t, dst_p, trash_p)
    dst_t = dst_t1.reshape(NW, NBE, KB)
    dsth = dst_t1.reshape(NW, HNB, HKB)
    dst_s = jnp.where(act_s, dst_p, trash_p).reshape(NW, NBE, KB)

    idx3 = jnp.stack([id_feature.astype(i32),
                      category_feature.astype(i32) + 256,
                      idx_feature.astype(i32) + 296])
    idx3 = jnp.pad(idx3, ((0, 0), (0, NP - N)))
    idx3 = idx3.reshape(3, NW, NBN, KGN).transpose(1, 0, 2, 3)

    bb8 = jnp.broadcast_to(
        jnp.pad(batch.astype(i32), (0, NP - N), constant_values=G)[:, None],
        (NP, 8))
    pf8 = jnp.pad(position_feature, ((0, NP - N), (0, 5)))
    pW8 = jnp.pad(pW, ((0, 5), (0, 0)))
    cat_emb_p = jnp.pad(cat_emb, ((0, 6), (0, 0)))
    idx_emb_p = jnp.pad(idx_emb, ((0, 4), (0, 0)))
    pfW8 = jnp.pad(pfW, ((0, 0), (0, 5)))
    pfb8 = jnp.pad(pfb, (0, 5))
    cfW128 = jnp.pad(cfW, ((0, 0), (0, D - 34)))
    cfb128 = jnp.pad(cfb, (0, D - 34), constant_values=NEG)

    ct = _tables(id_emb, cat_emb_p, idx_emb_p,
                 neW[128:256], neW[256:384], neW[384:512])
    gath, histp = _gath(ct, idx3, dsth)
    node, dinv8 = _node(pf8, pW8, pb[None], gath, neW[0:128], neb[None],
                        histp)

    pools = [_pool(node, bb8)]
    x = node
    for i in range(NLAYER):
        hs = _mm_scale(x, tW[i], dinv8)
        p = _spmm(hs, src_t, dst_t)
        gcn = _gcn_post(p, hs, dinv8, tb[i][None])
        q = _spmm(gcn, src_s, dst_s)
        nt = _gin(gcn, q, gW1[i], gb1[i][None], gW2[i], gb2[i][None])
        pools.append(_pool(nt, bb8))
        x = nt

    ge = jnp.concatenate(pools, axis=1)
    pos8, id_out, cat128 = _head(
        ge, agW, agb[None], pdW, pdb[None], pfW8, pfb8[None],
        idW, idb[None], ifW, ifb[None], cdW, cdb[None], cfW128, cfb128[None])
    return pos8[:, :3], id_out, cat128[:, :34]
